# baseline (speedup 1.0000x reference)
import functools

import jax
import jax.numpy as jnp
from jax import lax
from jax.experimental import pallas as pl
from jax.experimental.pallas import tpu as pltpu
from jax.experimental.pallas import tpu_sc as plsc

N_EXPERTS = 64
PD = 128
TB = 128
BR = 512
SC_CHUNK = 96


def _router_body(x_ref, wg_ref, bg_ref, p_ref, idx_ref):
    x = x_ref[...]
    wg = wg_ref[...]
    logits = lax.dot_general(x, wg, (((1,), (1,)), ((), ())),
                             preferred_element_type=jnp.float32)
    logits = logits + bg_ref[...]
    m = jnp.max(logits, axis=1, keepdims=True)
    ids = lax.broadcasted_iota(jnp.int32, logits.shape, 1)
    amax = jnp.min(jnp.where(logits == m, ids, N_EXPERTS), axis=1,
                   keepdims=True)
    denom = jnp.sum(jnp.exp(logits - m), axis=1, keepdims=True)
    p = 1.0 / denom
    p_ref[...] = jnp.broadcast_to(p, p_ref.shape)
    idx_ref[...] = jnp.broadcast_to(amax, idx_ref.shape)


def _route(x_flat, Wg, bg):
    n, d = x_flat.shape
    e = Wg.shape[0]
    return pl.pallas_call(
        _router_body,
        grid=(n // BR,),
        in_specs=[
            pl.BlockSpec((BR, d), lambda i: (i, 0)),
            pl.BlockSpec((e, d), lambda i: (0, 0)),
            pl.BlockSpec((1, e), lambda i: (0, 0)),
        ],
        out_specs=[
            pl.BlockSpec((BR, PD), lambda i: (i, 0)),
            pl.BlockSpec((BR, PD), lambda i: (i, 0)),
        ],
        out_shape=[
            jax.ShapeDtypeStruct((n, PD), jnp.float32),
            jax.ShapeDtypeStruct((n, PD), jnp.int32),
        ],
    )(x_flat, Wg, bg.reshape(1, e))


def _sc_dispatch(x_flat, p_rep, gperm, np_rows):
    n, d = x_flat.shape
    pd = p_rep.shape[1]
    info = plsc.get_sparse_core_info()
    nw = info.num_cores * info.num_subcores
    bpw = np_rows // nw
    nchunk = bpw // SC_CHUNK
    mesh = plsc.VectorSubcoreMesh(core_axis_name="c", subcore_axis_name="s")

    @functools.partial(
        pl.kernel,
        mesh=mesh,
        out_type=[
            jax.ShapeDtypeStruct((np_rows, d), jnp.float32),
            jax.ShapeDtypeStruct((np_rows, pd), jnp.float32),
        ],
        scratch_types=[
            pltpu.VMEM((SC_CHUNK,), jnp.int32),
            pltpu.VMEM((SC_CHUNK, d), jnp.float32),
            pltpu.VMEM((SC_CHUNK, pd), jnp.float32),
            pltpu.SemaphoreType.DMA,
        ],
    )
    def dispatch_k(x_hbm, p_hbm, pos_hbm, xs_hbm, ps_hbm, idx_v, xrow_v,
                   prow_v, sem):
        wid = lax.axis_index("s") * info.num_cores + lax.axis_index("c")
        base = wid * bpw
        for c in range(nchunk):
            cb = base + c * SC_CHUNK
            pltpu.sync_copy(pos_hbm.at[pl.ds(cb, SC_CHUNK)], idx_v)
            pltpu.async_copy(x_hbm.at[idx_v], xrow_v, sem).wait()
            pltpu.async_copy(p_hbm.at[idx_v], prow_v, sem).wait()
            pltpu.sync_copy(xrow_v, xs_hbm.at[pl.ds(cb, SC_CHUNK)])
            pltpu.sync_copy(prow_v, ps_hbm.at[pl.ds(cb, SC_CHUNK)])

    return dispatch_k(x_flat, p_rep, gperm)


def _sc_return(ys, aslot, perm):
    np_rows, d = ys.shape
    n = perm.shape[0]
    info = plsc.get_sparse_core_info()
    nw = info.num_cores * info.num_subcores
    bpw = n // nw
    mesh = plsc.VectorSubcoreMesh(core_axis_name="c", subcore_axis_name="s")

    @functools.partial(
        pl.kernel,
        mesh=mesh,
        out_type=jax.ShapeDtypeStruct((n, d), jnp.float32),
        scratch_types=[
            pltpu.VMEM((bpw,), jnp.int32),
            pltpu.VMEM((bpw,), jnp.int32),
            pltpu.VMEM((bpw, d), jnp.float32),
            pltpu.SemaphoreType.DMA,
        ],
    )
    def return_k(y_hbm, aslot_hbm, perm_hbm, out_hbm, i1_v, i2_v, row_v, sem):
        wid = lax.axis_index("s") * info.num_cores + lax.axis_index("c")
        base = wid * bpw
        pltpu.sync_copy(aslot_hbm.at[pl.ds(base, bpw)], i1_v)
        pltpu.sync_copy(perm_hbm.at[pl.ds(base, bpw)], i2_v)
        pltpu.async_copy(y_hbm.at[i1_v], row_v, sem).wait()
        pltpu.async_copy(row_v, out_hbm.at[i2_v], sem).wait()

    return return_k(ys, aslot, perm)


def _ffn_body(exp_r, en_r, xs_ref, ps_ref, w1_ref, b1_ref, w2_ref, b2_ref,
              out_ref):
    i = pl.program_id(0)
    x = xs_ref[...]
    h = lax.dot_general(x, w1_ref[0], (((1,), (1,)), ((), ())),
                        preferred_element_type=jnp.float32)
    h = jnp.maximum(h + b1_ref[0], 0.0)
    o = lax.dot_general(h, w2_ref[0], (((1,), (1,)), ((), ())),
                        preferred_element_type=jnp.float32)
    o = o + b2_ref[0]
    rows = i * TB + lax.broadcasted_iota(jnp.int32, (TB, 1), 0)
    scale = jnp.where(rows < en_r[i], ps_ref[:, 0:1], 0.0)
    out_ref[...] = o * scale


def _grouped_ffn(nblocks, entry_e, entry_t, xs, ps, W1, b1, W2, b2):
    np_rows, d = xs.shape
    e, f, _ = W1.shape
    grid_spec = pltpu.PrefetchScalarGridSpec(
        num_scalar_prefetch=2,
        grid=(nblocks,),
        in_specs=[
            pl.BlockSpec((TB, d), lambda i, ex, en: (i, 0)),
            pl.BlockSpec((TB, PD), lambda i, ex, en: (i, 0)),
            pl.BlockSpec((1, f, d), lambda i, ex, en: (ex[i], 0, 0)),
            pl.BlockSpec((1, 1, f), lambda i, ex, en: (ex[i], 0, 0)),
            pl.BlockSpec((1, d, f), lambda i, ex, en: (ex[i], 0, 0)),
            pl.BlockSpec((1, 1, d), lambda i, ex, en: (ex[i], 0, 0)),
        ],
        out_specs=pl.BlockSpec((TB, d), lambda i, ex, en: (i, 0)),
    )
    return pl.pallas_call(
        _ffn_body,
        grid_spec=grid_spec,
        out_shape=jax.ShapeDtypeStruct((np_rows, d), jnp.float32),
        compiler_params=pltpu.CompilerParams(
            dimension_semantics=("arbitrary",)),
    )(entry_e, entry_t,
      xs, ps, W1, b1.reshape(e, 1, f), W2, b2.reshape(e, 1, d))


def kernel(x, Wg, bg, W1, b1, W2, b2):
    batch, seq, d = x.shape
    n = batch * seq
    e_total = Wg.shape[0]
    x_flat = x.reshape(n, d)

    nw = 32
    gran = nw * SC_CHUNK
    np_rows = ((n + (e_total - 1) * TB + gran - 1) // gran) * gran
    ne_max = np_rows // TB

    p_rep, idx_rep = _route(x_flat, Wg, bg)
    eidx = idx_rep[:, 0]
    perm = jnp.argsort(eidx).astype(jnp.int32)
    counts = jnp.bincount(eidx, length=e_total).astype(jnp.int32)
    off = jnp.concatenate([jnp.zeros((1,), jnp.int32),
                           jnp.cumsum(counts).astype(jnp.int32)])

    pblk = (counts + TB - 1) // TB
    aoff_blk = jnp.concatenate([jnp.zeros((1,), jnp.int32),
                                jnp.cumsum(pblk).astype(jnp.int32)])
    aoff = aoff_blk * TB
    nblocks = aoff_blk[-1]

    i_arr = jnp.arange(ne_max, dtype=jnp.int32)
    e_i = jnp.searchsorted(aoff_blk[1:], i_arr, side="right").astype(jnp.int32)
    e_c = jnp.clip(e_i, 0, e_total - 1)
    entry_e = e_c
    entry_t = aoff[e_c] + counts[e_c]

    r_arr = jnp.arange(np_rows, dtype=jnp.int32)
    e_r = jnp.clip(jnp.searchsorted(aoff[1:], r_arr, side="right").astype(
        jnp.int32), 0, e_total - 1)
    within = r_arr - aoff[e_r]
    src = off[e_r] + jnp.minimum(within, jnp.maximum(counts[e_r] - 1, 0))
    gperm = perm[jnp.clip(src, 0, n - 1)]

    rn = jnp.arange(n, dtype=jnp.int32)
    e_s = jnp.clip(jnp.searchsorted(off[1:], rn, side="right").astype(
        jnp.int32), 0, e_total - 1)
    aslot = (aoff[e_s] + (rn - off[e_s])).astype(jnp.int32)

    xs, ps = _sc_dispatch(x_flat, p_rep, gperm, np_rows)
    ys = _grouped_ffn(nblocks, entry_e, entry_t, xs, ps, W1, b1, W2, b2)
    out_flat = _sc_return(ys, aslot, perm)
    return out_flat.reshape(batch, seq, d)


# speedup vs baseline: 5.3099x; 5.3099x over previous
import functools

import jax
import jax.numpy as jnp
from jax import lax
from jax.experimental import pallas as pl
from jax.experimental.pallas import tpu as pltpu
from jax.experimental.pallas import tpu_sc as plsc

N_EXPERTS = 64
PD = 128
TB = 256
BR = 512


def _router_body(x_ref, wg_ref, bg_ref, p_ref, idx_ref):
    x = x_ref[...]
    wg = wg_ref[...]
    logits = lax.dot_general(x, wg, (((1,), (1,)), ((), ())),
                             preferred_element_type=jnp.float32)
    logits = logits + bg_ref[...]
    m = jnp.max(logits, axis=1, keepdims=True)
    ids = lax.broadcasted_iota(jnp.int32, logits.shape, 1)
    amax = jnp.min(jnp.where(logits == m, ids, N_EXPERTS), axis=1,
                   keepdims=True)
    denom = jnp.sum(jnp.exp(logits - m), axis=1, keepdims=True)
    p = 1.0 / denom
    p_ref[...] = jnp.broadcast_to(p, p_ref.shape)
    idx_ref[...] = jnp.broadcast_to(amax, idx_ref.shape)


def _route(x_flat, Wg, bg):
    n, d = x_flat.shape
    e = Wg.shape[0]
    return pl.pallas_call(
        _router_body,
        grid=(n // BR,),
        in_specs=[
            pl.BlockSpec((BR, d), lambda i: (i, 0)),
            pl.BlockSpec((e, d), lambda i: (0, 0)),
            pl.BlockSpec((1, e), lambda i: (0, 0)),
        ],
        out_specs=[
            pl.BlockSpec((BR, PD), lambda i: (i, 0)),
            pl.BlockSpec((BR, PD), lambda i: (i, 0)),
        ],
        out_shape=[
            jax.ShapeDtypeStruct((n, PD), jnp.float32),
            jax.ShapeDtypeStruct((n, PD), jnp.int32),
        ],
    )(x_flat, Wg, bg.reshape(1, e))


def _sc_dispatch(x_flat, perm):
    n, d = x_flat.shape
    info = plsc.get_sparse_core_info()
    nw = info.num_cores * info.num_subcores
    bpw = n // nw
    mesh = plsc.VectorSubcoreMesh(core_axis_name="c", subcore_axis_name="s")

    @functools.partial(
        pl.kernel,
        mesh=mesh,
        out_type=jax.ShapeDtypeStruct((n, d), jnp.float32),
        scratch_types=[
            pltpu.VMEM((bpw,), jnp.int32),
            pltpu.VMEM((bpw, d), jnp.float32),
            pltpu.SemaphoreType.DMA,
        ],
    )
    def dispatch_k(x_hbm, pos_hbm, xs_hbm, idx_v, xrow_v, sem):
        wid = lax.axis_index("s") * info.num_cores + lax.axis_index("c")
        base = wid * bpw
        pltpu.sync_copy(pos_hbm.at[pl.ds(base, bpw)], idx_v)
        pltpu.async_copy(x_hbm.at[idx_v], xrow_v, sem).wait()
        pltpu.sync_copy(xrow_v, xs_hbm.at[pl.ds(base, bpw)])

    return dispatch_k(x_flat, perm)


def _sc_return(ys, perm):
    n, d = ys.shape
    info = plsc.get_sparse_core_info()
    nw = info.num_cores * info.num_subcores
    bpw = n // nw
    mesh = plsc.VectorSubcoreMesh(core_axis_name="c", subcore_axis_name="s")

    @functools.partial(
        pl.kernel,
        mesh=mesh,
        out_type=jax.ShapeDtypeStruct((n, d), jnp.float32),
        scratch_types=[
            pltpu.VMEM((bpw,), jnp.int32),
            pltpu.VMEM((bpw, d), jnp.float32),
            pltpu.SemaphoreType.DMA,
        ],
    )
    def return_k(y_hbm, pos_hbm, out_hbm, idx_v, row_v, sem):
        wid = lax.axis_index("s") * info.num_cores + lax.axis_index("c")
        base = wid * bpw
        pltpu.sync_copy(pos_hbm.at[pl.ds(base, bpw)], idx_v)
        pltpu.sync_copy(y_hbm.at[pl.ds(base, bpw)], row_v)
        pltpu.async_copy(row_v, out_hbm.at[idx_v], sem).wait()

    return return_k(ys, perm)


def _ffn_body(blk_r, exp_r, st_r, en_r, fst_r,
              xs_ref, w1_ref, b1_ref, w2_ref, b2_ref, out_ref):
    i = pl.program_id(0)

    @pl.when(fst_r[i] == 1)
    def _():
        out_ref[...] = jnp.zeros_like(out_ref)

    x = xs_ref[...]
    h = lax.dot_general(x, w1_ref[0], (((1,), (1,)), ((), ())),
                        preferred_element_type=jnp.float32)
    h = jnp.maximum(h + b1_ref[0], 0.0)
    o = lax.dot_general(h, w2_ref[0], (((1,), (1,)), ((), ())),
                        preferred_element_type=jnp.float32)
    o = o + b2_ref[0]
    rows = blk_r[i] * TB + lax.broadcasted_iota(jnp.int32, (TB, 1), 0)
    inseg = (rows >= st_r[i]) & (rows < en_r[i])
    out_ref[...] += jnp.where(inseg, o, 0.0)


def _grouped_ffn(entry_b, entry_e, entry_s, entry_t, entry_f,
                 xs, W1, b1, W2, b2, ne):
    n, d = xs.shape
    e, f, _ = W1.shape
    grid_spec = pltpu.PrefetchScalarGridSpec(
        num_scalar_prefetch=5,
        grid=(ne,),
        in_specs=[
            pl.BlockSpec((TB, d), lambda i, b, ex, s, t, fr: (b[i], 0)),
            pl.BlockSpec((1, f, d), lambda i, b, ex, s, t, fr: (ex[i], 0, 0)),
            pl.BlockSpec((1, 1, f), lambda i, b, ex, s, t, fr: (ex[i], 0, 0)),
            pl.BlockSpec((1, d, f), lambda i, b, ex, s, t, fr: (ex[i], 0, 0)),
            pl.BlockSpec((1, 1, d), lambda i, b, ex, s, t, fr: (ex[i], 0, 0)),
        ],
        out_specs=pl.BlockSpec((TB, d), lambda i, b, ex, s, t, fr: (b[i], 0)),
    )
    return pl.pallas_call(
        _ffn_body,
        grid_spec=grid_spec,
        out_shape=jax.ShapeDtypeStruct((n, d), jnp.float32),
        compiler_params=pltpu.CompilerParams(
            dimension_semantics=("arbitrary",)),
    )(entry_b, entry_e, entry_s, entry_t, entry_f,
      xs, W1, b1.reshape(e, 1, f), W2, b2.reshape(e, 1, d))


def _build_entries(eidx, counts, off, e_total, nb):
    ne = nb + e_total - 1
    first_blk = off[:e_total] // TB
    last_blk = (off[1:] - 1) // TB
    n_e = jnp.where(counts > 0, last_blk - first_blk + 1, 0).astype(jnp.int32)
    cum = jnp.cumsum(n_e).astype(jnp.int32)
    starts = cum - n_e
    r_total = cum[-1]
    i_arr = jnp.arange(ne, dtype=jnp.int32)
    e_i = jnp.searchsorted(cum, i_arr, side="right").astype(jnp.int32)
    valid = i_arr < r_total
    e_c = jnp.clip(e_i, 0, e_total - 1)
    pad_e = jnp.max(eidx).astype(jnp.int32)
    entry_e = jnp.where(valid, e_c, pad_e)
    entry_b = jnp.where(valid, first_blk[e_c] + (i_arr - starts[e_c]), nb - 1)
    entry_s = jnp.where(valid, off[e_c], 0)
    entry_t = jnp.where(valid, off[e_c + 1], 0)
    prev_b = jnp.concatenate([jnp.full((1,), -1, jnp.int32), entry_b[:-1]])
    entry_f = (entry_b != prev_b).astype(jnp.int32)
    return entry_b, entry_e, entry_s, entry_t, entry_f, ne


def kernel(x, Wg, bg, W1, b1, W2, b2):
    batch, seq, d = x.shape
    n = batch * seq
    nb = n // TB
    e_total = Wg.shape[0]
    x_flat = x.reshape(n, d)

    p_rep, idx_rep = _route(x_flat, Wg, bg)
    eidx = idx_rep[:, 0]
    perm = jnp.argsort(eidx).astype(jnp.int32)
    counts = jnp.bincount(eidx, length=e_total)
    off = jnp.concatenate([jnp.zeros((1,), jnp.int32),
                           jnp.cumsum(counts).astype(jnp.int32)])

    entry_b, entry_e, entry_s, entry_t, entry_f, ne = _build_entries(
        eidx, counts, off, e_total, nb)

    xs = _sc_dispatch(x_flat, perm)
    ys = _grouped_ffn(entry_b, entry_e, entry_s, entry_t, entry_f,
                      xs, W1, b1, W2, b2, ne)
    out_flat = _sc_return(ys, perm) * p_rep[:, 0:1]
    return out_flat.reshape(batch, seq, d)


# speedup vs baseline: 5.4484x; 1.0261x over previous
import functools

import jax
import jax.numpy as jnp
from jax import lax
from jax.experimental import pallas as pl
from jax.experimental.pallas import tpu as pltpu
from jax.experimental.pallas import tpu_sc as plsc

N_EXPERTS = 64
PD = 128
TB = 256
BR = 512


def _router_body(x_ref, wg_ref, bg_ref, p_ref, idx_ref):
    x = x_ref[...]
    wg = wg_ref[...]
    logits = lax.dot_general(x, wg, (((1,), (1,)), ((), ())),
                             preferred_element_type=jnp.float32)
    logits = logits + bg_ref[...]
    m = jnp.max(logits, axis=1, keepdims=True)
    ids = lax.broadcasted_iota(jnp.int32, logits.shape, 1)
    amax = jnp.min(jnp.where(logits == m, ids, N_EXPERTS), axis=1,
                   keepdims=True)
    denom = jnp.sum(jnp.exp(logits - m), axis=1, keepdims=True)
    p = 1.0 / denom
    p_ref[...] = jnp.broadcast_to(p, p_ref.shape)
    idx_ref[...] = jnp.broadcast_to(amax, idx_ref.shape)


def _route(x_flat, Wg, bg):
    n, d = x_flat.shape
    e = Wg.shape[0]
    return pl.pallas_call(
        _router_body,
        grid=(n // BR,),
        in_specs=[
            pl.BlockSpec((BR, d), lambda i: (i, 0)),
            pl.BlockSpec((e, d), lambda i: (0, 0)),
            pl.BlockSpec((1, e), lambda i: (0, 0)),
        ],
        out_specs=[
            pl.BlockSpec((BR, PD), lambda i: (i, 0)),
            pl.BlockSpec((BR, PD), lambda i: (i, 0)),
        ],
        out_shape=[
            jax.ShapeDtypeStruct((n, PD), jnp.float32),
            jax.ShapeDtypeStruct((n, PD), jnp.int32),
        ],
    )(x_flat, Wg, bg.reshape(1, e))


def _sc_dispatch(x_flat, p_rep, perm):
    n, d = x_flat.shape
    pd = p_rep.shape[1]
    info = plsc.get_sparse_core_info()
    nw = info.num_cores * info.num_subcores
    bpw = n // nw
    mesh = plsc.VectorSubcoreMesh(core_axis_name="c", subcore_axis_name="s")

    @functools.partial(
        pl.kernel,
        mesh=mesh,
        out_type=[
            jax.ShapeDtypeStruct((n, d), jnp.float32),
            jax.ShapeDtypeStruct((n, pd), jnp.float32),
        ],
        scratch_types=[
            pltpu.VMEM((bpw,), jnp.int32),
            pltpu.VMEM((bpw, d), jnp.float32),
            pltpu.VMEM((bpw, pd), jnp.float32),
            pltpu.SemaphoreType.DMA,
        ],
    )
    def dispatch_k(x_hbm, p_hbm, pos_hbm, xs_hbm, ps_hbm, idx_v, xrow_v,
                   prow_v, sem):
        wid = lax.axis_index("s") * info.num_cores + lax.axis_index("c")
        base = wid * bpw
        pltpu.sync_copy(pos_hbm.at[pl.ds(base, bpw)], idx_v)
        pltpu.async_copy(x_hbm.at[idx_v], xrow_v, sem).wait()
        pltpu.async_copy(p_hbm.at[idx_v], prow_v, sem).wait()
        pltpu.sync_copy(xrow_v, xs_hbm.at[pl.ds(base, bpw)])
        pltpu.sync_copy(prow_v, ps_hbm.at[pl.ds(base, bpw)])

    return dispatch_k(x_flat, p_rep, perm)


def _sc_return(ys, perm):
    n, d = ys.shape
    info = plsc.get_sparse_core_info()
    nw = info.num_cores * info.num_subcores
    bpw = n // nw
    mesh = plsc.VectorSubcoreMesh(core_axis_name="c", subcore_axis_name="s")

    @functools.partial(
        pl.kernel,
        mesh=mesh,
        out_type=jax.ShapeDtypeStruct((n, d), jnp.float32),
        scratch_types=[
            pltpu.VMEM((bpw,), jnp.int32),
            pltpu.VMEM((bpw, d), jnp.float32),
            pltpu.SemaphoreType.DMA,
        ],
    )
    def return_k(y_hbm, pos_hbm, out_hbm, idx_v, row_v, sem):
        wid = lax.axis_index("s") * info.num_cores + lax.axis_index("c")
        base = wid * bpw
        pltpu.sync_copy(pos_hbm.at[pl.ds(base, bpw)], idx_v)
        pltpu.sync_copy(y_hbm.at[pl.ds(base, bpw)], row_v)
        pltpu.async_copy(row_v, out_hbm.at[idx_v], sem).wait()

    return return_k(ys, perm)


def _ffn_body(blk_r, exp_r, st_r, en_r, fst_r,
              xs_ref, ps_ref, w1_ref, b1_ref, w2_ref, b2_ref, out_ref):
    i = pl.program_id(0)

    @pl.when(fst_r[i] == 1)
    def _():
        out_ref[...] = jnp.zeros_like(out_ref)

    x = xs_ref[...].astype(jnp.bfloat16)
    w1 = w1_ref[0].astype(jnp.bfloat16)
    h = lax.dot_general(x, w1, (((1,), (1,)), ((), ())),
                        preferred_element_type=jnp.float32)
    h = jnp.maximum(h + b1_ref[0], 0.0)
    w2 = w2_ref[0].astype(jnp.bfloat16)
    o = lax.dot_general(h.astype(jnp.bfloat16), w2, (((1,), (1,)), ((), ())),
                        preferred_element_type=jnp.float32)
    o = o + b2_ref[0]
    rows = blk_r[i] * TB + lax.broadcasted_iota(jnp.int32, (TB, 1), 0)
    inseg = (rows >= st_r[i]) & (rows < en_r[i])
    scale = jnp.where(inseg, ps_ref[:, 0:1], 0.0)
    out_ref[...] += o * scale


def _grouped_ffn(entry_b, entry_e, entry_s, entry_t, entry_f,
                 xs, ps, W1, b1, W2, b2, ne):
    n, d = xs.shape
    e, f, _ = W1.shape
    grid_spec = pltpu.PrefetchScalarGridSpec(
        num_scalar_prefetch=5,
        grid=(ne,),
        in_specs=[
            pl.BlockSpec((TB, d), lambda i, b, ex, s, t, fr: (b[i], 0)),
            pl.BlockSpec((TB, PD), lambda i, b, ex, s, t, fr: (b[i], 0)),
            pl.BlockSpec((1, f, d), lambda i, b, ex, s, t, fr: (ex[i], 0, 0)),
            pl.BlockSpec((1, 1, f), lambda i, b, ex, s, t, fr: (ex[i], 0, 0)),
            pl.BlockSpec((1, d, f), lambda i, b, ex, s, t, fr: (ex[i], 0, 0)),
            pl.BlockSpec((1, 1, d), lambda i, b, ex, s, t, fr: (ex[i], 0, 0)),
        ],
        out_specs=pl.BlockSpec((TB, d), lambda i, b, ex, s, t, fr: (b[i], 0)),
    )
    return pl.pallas_call(
        _ffn_body,
        grid_spec=grid_spec,
        out_shape=jax.ShapeDtypeStruct((n, d), jnp.float32),
        compiler_params=pltpu.CompilerParams(
            dimension_semantics=("arbitrary",)),
    )(entry_b, entry_e, entry_s, entry_t, entry_f,
      xs, ps, W1, b1.reshape(e, 1, f), W2, b2.reshape(e, 1, d))


def _build_entries(eidx, counts, off, e_total, nb):
    ne = nb + e_total - 1
    first_blk = off[:e_total] // TB
    last_blk = (off[1:] - 1) // TB
    n_e = jnp.where(counts > 0, last_blk - first_blk + 1, 0).astype(jnp.int32)
    cum = jnp.cumsum(n_e).astype(jnp.int32)
    starts = cum - n_e
    r_total = cum[-1]
    i_arr = jnp.arange(ne, dtype=jnp.int32)
    e_i = jnp.searchsorted(cum, i_arr, side="right").astype(jnp.int32)
    valid = i_arr < r_total
    e_c = jnp.clip(e_i, 0, e_total - 1)
    pad_e = jnp.max(eidx).astype(jnp.int32)
    entry_e = jnp.where(valid, e_c, pad_e)
    entry_b = jnp.where(valid, first_blk[e_c] + (i_arr - starts[e_c]), nb - 1)
    entry_s = jnp.where(valid, off[e_c], 0)
    entry_t = jnp.where(valid, off[e_c + 1], 0)
    prev_b = jnp.concatenate([jnp.full((1,), -1, jnp.int32), entry_b[:-1]])
    entry_f = (entry_b != prev_b).astype(jnp.int32)
    return entry_b, entry_e, entry_s, entry_t, entry_f, ne


def kernel(x, Wg, bg, W1, b1, W2, b2):
    batch, seq, d = x.shape
    n = batch * seq
    nb = n // TB
    e_total = Wg.shape[0]
    x_flat = x.reshape(n, d)

    p_rep, idx_rep = _route(x_flat, Wg, bg)
    eidx = idx_rep[:, 0]
    perm = jnp.argsort(eidx).astype(jnp.int32)
    counts = jnp.bincount(eidx, length=e_total)
    off = jnp.concatenate([jnp.zeros((1,), jnp.int32),
                           jnp.cumsum(counts).astype(jnp.int32)])

    entry_b, entry_e, entry_s, entry_t, entry_f, ne = _build_entries(
        eidx, counts, off, e_total, nb)

    xs, ps = _sc_dispatch(x_flat, p_rep, perm)
    ys = _grouped_ffn(entry_b, entry_e, entry_s, entry_t, entry_f,
                      xs, ps, W1, b1, W2, b2, ne)
    out_flat = _sc_return(ys, perm)
    return out_flat.reshape(batch, seq, d)
